# baseline (speedup 1.0000x reference)
import functools

import jax
import jax.numpy as jnp
from jax import lax
from jax.experimental import pallas as pl
from jax.experimental.pallas import tpu as pltpu
from jax.experimental.pallas import tpu_sc as plsc

NV = 100000
NF = 200000
CIN = 128
COUT = 128
NB = 16

NPASS = 4
NCORE = 2
NSUB = 16
SPAN = 12544
ACC_ROWS = 12672
NV_PAD = NPASS * NCORE * SPAN
COL_LEN = 212992
COL_W = COL_LEN // NSUB
EB = 1024
NEB = COL_W // EB
G = 64
CF_CAP = 2 * EB + 3 * G
WB_ROWS = SPAN // NSUB
ZB_ROWS = ACC_ROWS // NSUB


def _sc_scatter_body(ev_hbm, filt_hbm, zero_hbm, agg_hbm,
                     acc, cf, evbA, evbB,
                     cfcA, ddcA, cfcB, ddcB, cfcC, ddcC,
                     rowsA, rowsB, rowsC,
                     gsemA, gsemB, gsemC, ssemA, ssemB, ssemC, esemA, esemB):
    c = lax.axis_index("c")
    s = lax.axis_index("s")
    iota16 = lax.iota(jnp.int32, 16)

    pad16 = jnp.full((16,), ((SPAN + 0) << 18), jnp.int32) + (
        jnp.full((16,), s, jnp.int32) * ((1 << 18) + 1))

    def stage_idx(base, cfcX, ddcX):
        mask_f = jnp.full((16,), (1 << 18) - 1, jnp.int32)
        for t in range(G // 16):
            packed = cf[pl.ds(base + t * 16, 16)]
            cfcX[pl.ds(t * 16, 16)] = packed & mask_f
            ddcX[pl.ds(t * 16, 16)] = lax.shift_right_logical(packed, 18)

    def start_gather(cfcX, rowsX, gsemX):
        pltpu.async_copy(filt_hbm.at[cfcX], rowsX, gsemX)

    def wait_gather(cfcX, rowsX, gsemX):
        pltpu.make_async_copy(filt_hbm.at[cfcX], rowsX, gsemX).wait()

    def start_scatter(rowsX, ddcX, ssemX):
        pltpu.async_copy(rowsX, acc.at[ddcX], ssemX, add=True)

    def wait_scatter(rowsX, ddcX, ssemX):
        pltpu.make_async_copy(rowsX, acc.at[ddcX], ssemX).wait()

    def flush(cnt):
        for j in range(3 * G // 16):
            cf[pl.ds(cnt + j * 16, 16)] = pad16
        nch3 = ((cnt + (3 * G - 1)) // (3 * G)) * 3

        def pipe(nch3):
            BUFS = [(cfcA, ddcA, rowsA, gsemA, ssemA),
                    (cfcB, ddcB, rowsB, gsemB, ssemB),
                    (cfcC, ddcC, rowsC, gsemC, ssemC)]
            stage_idx(0, cfcA, ddcA)
            start_gather(cfcA, rowsA, gsemA)

            def triple(j, carry):
                for i in range(3):
                    X = BUFS[i]
                    W = BUFS[(i + 1) % 3]
                    kk = 3 * j + i

                    @pl.when(kk + 1 < nch3)
                    def _(X=X, W=W, kk=kk):
                        @pl.when(kk + 1 >= 3)
                        def _():
                            wait_scatter(W[2], W[1], W[4])
                        stage_idx((kk + 1) * G, W[0], W[1])
                        start_gather(W[0], W[2], W[3])
                    wait_gather(X[0], X[2], X[3])
                    start_scatter(X[2], X[1], X[4])
                return carry

            lax.fori_loop(0, nch3 // 3, triple, 0)
            for i in range(3):
                X = BUFS[i]
                wait_scatter(X[2], X[1], X[4])
            return jnp.int32(0)

        return lax.cond(nch3 > 0, pipe, lambda n: jnp.int32(0), nch3)

    def start_estage(col, b, evbX, esemX):
        off = col * COL_LEN + s * COL_W + b * EB
        pltpu.async_copy(ev_hbm.at[pl.ds(off, EB)], evbX, esemX)

    def wait_estage(col, b, evbX, esemX):
        off = col * COL_LEN + s * COL_W + b * EB
        pltpu.make_async_copy(ev_hbm.at[pl.ds(off, EB)], evbX, esemX).wait()

    def pass_body(p, _):
        lo = (2 * p + c) * SPAN

        pltpu.sync_copy(zero_hbm.at[pl.ds(s * ZB_ROWS, ZB_ROWS), :],
                        acc.at[pl.ds(s * ZB_ROWS, ZB_ROWS), :])
        plsc.subcore_barrier()

        def compact(evbX, fbase, cnt):
            def t32(t, cnt):
                v1 = evbX[pl.ds(t * 32, 16)]
                v2 = evbX[pl.ds(t * 32 + 16, 16)]
                f1 = fbase + t * 32 + iota16
                f2 = f1 + 16
                d1 = v1 - lo
                d2 = v2 - lo
                m1 = (d1 >= 0) & (d1 < SPAN)
                m2 = (d2 >= 0) & (d2 < SPAN)
                cs1 = plsc.cumsum(m1.astype(jnp.int32))
                cs2 = plsc.cumsum(m2.astype(jnp.int32))
                p1 = (d1 << 18) | f1
                p2 = (d2 << 18) | f2
                pc1 = plsc.all_reduce_population_count(m1)
                pc2 = plsc.all_reduce_population_count(m2)
                plsc.store_scatter(cf, [cnt + cs1 - 1], p1, mask=m1)
                cnt1 = cnt + pc1[0]
                plsc.store_scatter(cf, [cnt1 + cs2 - 1], p2, mask=m2)
                return cnt1 + pc2[0]

            cnt = lax.fori_loop(0, EB // 32, t32, cnt)
            return lax.cond(cnt >= EB, flush, lambda x: x, cnt)

        k_cnt = jnp.int32(0)
        for col in range(3):
            fcol = s * COL_W
            start_estage(col, 0, evbA, esemA)

            def pairblk(q, cnt, col=col, fcol=fcol):
                start_estage(col, 2 * q + 1, evbB, esemB)
                wait_estage(col, 2 * q, evbA, esemA)
                cnt = compact(evbA, fcol + 2 * q * EB, cnt)
                start_estage(col, 2 * q + 2, evbA, esemA)
                wait_estage(col, 2 * q + 1, evbB, esemB)
                return compact(evbB, fcol + (2 * q + 1) * EB, cnt)

            k_cnt = lax.fori_loop(0, NEB // 2, pairblk, k_cnt)
            wait_estage(col, NEB - 1, evbA, esemA)
            k_cnt = compact(evbA, fcol + (NEB - 1) * EB, k_cnt)
        flush(k_cnt)
        plsc.subcore_barrier()

        pltpu.sync_copy(acc.at[pl.ds(s * WB_ROWS, WB_ROWS), :],
                        agg_hbm.at[pl.ds(lo + s * WB_ROWS, WB_ROWS), :])
        plsc.subcore_barrier()
        return 0

    lax.fori_loop(0, NPASS, pass_body, 0)


def _sc_scatter(ev, filtered, zeros):
    mesh = plsc.VectorSubcoreMesh(core_axis_name="c", subcore_axis_name="s")
    return pl.kernel(
        _sc_scatter_body,
        out_type=jax.ShapeDtypeStruct((NV_PAD, CIN), jnp.float32),
        mesh=mesh,
        scratch_types=[
            pltpu.VMEM_SHARED((ACC_ROWS, CIN), jnp.float32),
            pltpu.VMEM((CF_CAP,), jnp.int32),
            pltpu.VMEM((EB,), jnp.int32),
            pltpu.VMEM((EB,), jnp.int32),
            pltpu.VMEM((G,), jnp.int32),
            pltpu.VMEM((G,), jnp.int32),
            pltpu.VMEM((G,), jnp.int32),
            pltpu.VMEM((G,), jnp.int32),
            pltpu.VMEM((G,), jnp.int32),
            pltpu.VMEM((G,), jnp.int32),
            pltpu.VMEM((G, CIN), jnp.float32),
            pltpu.VMEM((G, CIN), jnp.float32),
            pltpu.VMEM((G, CIN), jnp.float32),
            pltpu.SemaphoreType.DMA,
            pltpu.SemaphoreType.DMA,
            pltpu.SemaphoreType.DMA,
            pltpu.SemaphoreType.DMA,
            pltpu.SemaphoreType.DMA,
            pltpu.SemaphoreType.DMA,
            pltpu.SemaphoreType.DMA,
            pltpu.SemaphoreType.DMA,
        ],
        compiler_params=pltpu.CompilerParams(needs_layout_passes=False),
    )(ev, filtered, zeros)


def _tc1_body(fc_ref, x_ref, sw_ref, o_ref):
    o_ref[...] = (
        jnp.dot(fc_ref[...], sw_ref[...], preferred_element_type=jnp.float32)
        * x_ref[...]
    )


def _tc1(filt_coeff, inputs, sw):
    blk = 4000
    grid = (NF // blk,)
    return pl.pallas_call(
        _tc1_body,
        grid=grid,
        in_specs=[
            pl.BlockSpec((blk, NB), lambda i: (i, 0)),
            pl.BlockSpec((blk, CIN), lambda i: (i, 0)),
            pl.BlockSpec((NB, CIN), lambda i: (0, 0)),
        ],
        out_specs=pl.BlockSpec((blk, CIN), lambda i: (i, 0)),
        out_shape=jax.ShapeDtypeStruct((NF, CIN), jnp.float32),
    )(filt_coeff, inputs, sw)


def _tc3a_body(agg_ref, nfc_ref, dw_ref, b_ref, s_ref):
    den = jnp.maximum(nfc_ref[...], 1).astype(jnp.float32)
    x = agg_ref[...] / den
    y = jnp.maximum(
        jnp.dot(x, dw_ref[...], preferred_element_type=jnp.float32) + b_ref[...],
        0.0,
    )
    ps = jnp.concatenate(
        [jnp.sum(y, axis=0)[None, :], jnp.sum(y * y, axis=0)[None, :],
         jnp.zeros((6, COUT), jnp.float32)], axis=0)

    @pl.when(pl.program_id(0) == 0)
    def _():
        s_ref[...] = ps

    @pl.when(pl.program_id(0) > 0)
    def _():
        s_ref[...] += ps


def _tc3a(agg, nf_count, dw, b):
    blk = 5000
    grid = (NV // blk,)
    return pl.pallas_call(
        _tc3a_body,
        grid=grid,
        in_specs=[
            pl.BlockSpec((blk, CIN), lambda i: (i, 0)),
            pl.BlockSpec((blk, 1), lambda i: (i, 0)),
            pl.BlockSpec((CIN, COUT), lambda i: (0, 0)),
            pl.BlockSpec((1, COUT), lambda i: (0, 0)),
        ],
        out_specs=pl.BlockSpec((8, COUT), lambda i: (0, 0)),
        out_shape=jax.ShapeDtypeStruct((8, COUT), jnp.float32),
    )(agg, nf_count.reshape(NV, 1), dw, b)


def _tc3b_body(agg_ref, nfc_ref, dw_ref, b_ref, s_ref, g_ref, bb_ref, o_ref):
    den = jnp.maximum(nfc_ref[...], 1).astype(jnp.float32)
    x = agg_ref[...] / den
    y = jnp.maximum(
        jnp.dot(x, dw_ref[...], preferred_element_type=jnp.float32) + b_ref[...],
        0.0,
    )
    inv_n = jnp.float32(1.0 / NV)
    mean = s_ref[0, :] * inv_n
    var = s_ref[1, :] * inv_n - mean * mean
    scale = g_ref[0, :] * lax.rsqrt(var + jnp.float32(1e-3))
    o_ref[...] = y * scale + (bb_ref[0, :] - mean * scale)


def _tc3b(agg, nf_count, dw, b, sums, gamma, beta):
    blk = 5000
    grid = (NV // blk,)
    return pl.pallas_call(
        _tc3b_body,
        grid=grid,
        in_specs=[
            pl.BlockSpec((blk, CIN), lambda i: (i, 0)),
            pl.BlockSpec((blk, 1), lambda i: (i, 0)),
            pl.BlockSpec((CIN, COUT), lambda i: (0, 0)),
            pl.BlockSpec((1, COUT), lambda i: (0, 0)),
            pl.BlockSpec((8, COUT), lambda i: (0, 0)),
            pl.BlockSpec((1, COUT), lambda i: (0, 0)),
            pl.BlockSpec((1, COUT), lambda i: (0, 0)),
        ],
        out_specs=pl.BlockSpec((blk, COUT), lambda i: (i, 0)),
        out_shape=jax.ShapeDtypeStruct((NV, COUT), jnp.float32),
    )(agg, nf_count.reshape(NV, 1), dw, b, sums, gamma, beta)


def kernel(inputs, face, nf_count, vt_map, filt_coeff, spatial_weights,
           depth_weights, biases, bn_gamma, bn_beta):
    del vt_map
    sw = spatial_weights.reshape(NB, CIN)
    filtered = _tc1(filt_coeff, inputs, sw)

    ev = jnp.concatenate(
        [face.T, jnp.full((3, COL_LEN - NF), jnp.int32(1 << 30), jnp.int32)],
        axis=1).reshape(-1)

    zeros = jnp.zeros((ACC_ROWS, CIN), jnp.float32)
    agg_pad = _sc_scatter(ev, filtered, zeros)

    sums = _tc3a(agg_pad, nf_count, depth_weights, biases)
    out = _tc3b(agg_pad, nf_count, depth_weights, biases, sums,
                bn_gamma.reshape(1, COUT), bn_beta.reshape(1, COUT))
    return out


# speedup vs baseline: 1.0089x; 1.0089x over previous
import jax
import jax.numpy as jnp
from jax import lax
from jax.experimental import pallas as pl
from jax.experimental.pallas import tpu as pltpu
from jax.experimental.pallas import tpu_sc as plsc

NV = 100000
NF = 200000
CIN = 128
COUT = 128
NB = 16

NPASS = 4
NCORE = 2
NSUB = 16
SPAN = 12544
ACC_ROWS = 12672
NV_PAD = NPASS * NCORE * SPAN
COL_LEN = 212992
COL_W = COL_LEN // NSUB
EB = 1024
NEB = COL_W // EB
G = 64
CF_CAP = 2 * EB + 2 * G
WB_ROWS = SPAN // NSUB
ZB_ROWS = ACC_ROWS // NSUB


def _sc_scatter_body(ev_hbm, filt_hbm, zero_hbm, agg_hbm,
                     acc, cf, evbA, evbB,
                     cfcA, ddcA, cfcB, ddcB, rowsA, rowsB,
                     gsemA, gsemB, ssemA, ssemB, esemA, esemB):
    c = lax.axis_index("c")
    s = lax.axis_index("s")
    iota16 = lax.iota(jnp.int32, 16)

    pad16 = jnp.full((16,), ((SPAN + 0) << 18), jnp.int32) + (
        jnp.full((16,), s, jnp.int32) * ((1 << 18) + 1))

    def stage_idx(base, cfcX, ddcX):
        mask_f = jnp.full((16,), (1 << 18) - 1, jnp.int32)
        for t in range(G // 16):
            packed = cf[pl.ds(base + t * 16, 16)]
            cfcX[pl.ds(t * 16, 16)] = packed & mask_f
            ddcX[pl.ds(t * 16, 16)] = lax.shift_right_logical(packed, 18)

    def start_gather(cfcX, rowsX, gsemX):
        pltpu.async_copy(filt_hbm.at[cfcX], rowsX, gsemX)

    def wait_gather(cfcX, rowsX, gsemX):
        pltpu.make_async_copy(filt_hbm.at[cfcX], rowsX, gsemX).wait()

    def start_scatter(rowsX, ddcX, ssemX):
        pltpu.async_copy(rowsX, acc.at[ddcX], ssemX, add=True)

    def wait_scatter(rowsX, ddcX, ssemX):
        pltpu.make_async_copy(rowsX, acc.at[ddcX], ssemX).wait()

    def flush(cnt):
        for j in range(2 * G // 16):
            cf[pl.ds(cnt + j * 16, 16)] = pad16
        npairs = (cnt + (2 * G - 1)) // (2 * G)

        def pipe(npairs):
            stage_idx(0, cfcA, ddcA)
            start_gather(cfcA, rowsA, gsemA)

            def pair(j, carry):
                @pl.when(j > 0)
                def _():
                    wait_scatter(rowsB, ddcB, ssemB)
                stage_idx(j * 2 * G + G, cfcB, ddcB)
                start_gather(cfcB, rowsB, gsemB)
                wait_gather(cfcA, rowsA, gsemA)
                start_scatter(rowsA, ddcA, ssemA)
                wait_scatter(rowsA, ddcA, ssemA)

                @pl.when(j + 1 < npairs)
                def _():
                    stage_idx((j + 1) * 2 * G, cfcA, ddcA)
                    start_gather(cfcA, rowsA, gsemA)
                wait_gather(cfcB, rowsB, gsemB)
                start_scatter(rowsB, ddcB, ssemB)
                return carry

            lax.fori_loop(0, npairs, pair, 0)
            wait_scatter(rowsB, ddcB, ssemB)
            return jnp.int32(0)

        return lax.cond(npairs > 0, pipe, lambda n: jnp.int32(0), npairs)

    def start_estage(col, b, evbX, esemX):
        off = col * COL_LEN + s * COL_W + b * EB
        pltpu.async_copy(ev_hbm.at[pl.ds(off, EB)], evbX, esemX)

    def wait_estage(col, b, evbX, esemX):
        off = col * COL_LEN + s * COL_W + b * EB
        pltpu.make_async_copy(ev_hbm.at[pl.ds(off, EB)], evbX, esemX).wait()

    def pass_body(p, _):
        lo = (2 * p + c) * SPAN

        pltpu.sync_copy(zero_hbm.at[pl.ds(s * ZB_ROWS, ZB_ROWS), :],
                        acc.at[pl.ds(s * ZB_ROWS, ZB_ROWS), :])
        plsc.subcore_barrier()

        def compact(evbX, fbase, cnt):
            def t32(t, cnt):
                v1 = evbX[pl.ds(t * 32, 16)]
                v2 = evbX[pl.ds(t * 32 + 16, 16)]
                f1 = fbase + t * 32 + iota16
                f2 = f1 + 16
                d1 = v1 - lo
                d2 = v2 - lo
                m1 = (d1 >= 0) & (d1 < SPAN)
                m2 = (d2 >= 0) & (d2 < SPAN)
                cs1 = plsc.cumsum(m1.astype(jnp.int32))
                cs2 = plsc.cumsum(m2.astype(jnp.int32))
                p1 = (d1 << 18) | f1
                p2 = (d2 << 18) | f2
                pc1 = plsc.all_reduce_population_count(m1)
                pc2 = plsc.all_reduce_population_count(m2)
                plsc.store_scatter(cf, [cnt + cs1 - 1], p1, mask=m1)
                cnt1 = cnt + pc1[0]
                plsc.store_scatter(cf, [cnt1 + cs2 - 1], p2, mask=m2)
                return cnt1 + pc2[0]

            cnt = lax.fori_loop(0, EB // 32, t32, cnt)
            return lax.cond(cnt >= EB, flush, lambda x: x, cnt)

        k_cnt = jnp.int32(0)
        for col in range(3):
            fcol = s * COL_W
            start_estage(col, 0, evbA, esemA)

            def pairblk(q, cnt, col=col, fcol=fcol):
                start_estage(col, 2 * q + 1, evbB, esemB)
                wait_estage(col, 2 * q, evbA, esemA)
                cnt = compact(evbA, fcol + 2 * q * EB, cnt)
                start_estage(col, 2 * q + 2, evbA, esemA)
                wait_estage(col, 2 * q + 1, evbB, esemB)
                return compact(evbB, fcol + (2 * q + 1) * EB, cnt)

            k_cnt = lax.fori_loop(0, NEB // 2, pairblk, k_cnt)
            wait_estage(col, NEB - 1, evbA, esemA)
            k_cnt = compact(evbA, fcol + (NEB - 1) * EB, k_cnt)
        flush(k_cnt)
        plsc.subcore_barrier()

        pltpu.sync_copy(acc.at[pl.ds(s * WB_ROWS, WB_ROWS), :],
                        agg_hbm.at[pl.ds(lo + s * WB_ROWS, WB_ROWS), :])
        plsc.subcore_barrier()
        return 0

    lax.fori_loop(0, NPASS, pass_body, 0)


def _sc_scatter(ev, filtered, zeros):
    mesh = plsc.VectorSubcoreMesh(core_axis_name="c", subcore_axis_name="s")
    return pl.kernel(
        _sc_scatter_body,
        out_type=jax.ShapeDtypeStruct((NV_PAD, CIN), jnp.float32),
        mesh=mesh,
        scratch_types=[
            pltpu.VMEM_SHARED((ACC_ROWS, CIN), jnp.float32),
            pltpu.VMEM((CF_CAP,), jnp.int32),
            pltpu.VMEM((EB,), jnp.int32),
            pltpu.VMEM((EB,), jnp.int32),
            pltpu.VMEM((G,), jnp.int32),
            pltpu.VMEM((G,), jnp.int32),
            pltpu.VMEM((G,), jnp.int32),
            pltpu.VMEM((G,), jnp.int32),
            pltpu.VMEM((G, CIN), jnp.float32),
            pltpu.VMEM((G, CIN), jnp.float32),
            pltpu.SemaphoreType.DMA,
            pltpu.SemaphoreType.DMA,
            pltpu.SemaphoreType.DMA,
            pltpu.SemaphoreType.DMA,
            pltpu.SemaphoreType.DMA,
            pltpu.SemaphoreType.DMA,
        ],
        compiler_params=pltpu.CompilerParams(needs_layout_passes=False),
    )(ev, filtered, zeros)


def _tc1_body(fc_ref, x_ref, sw_ref, o_ref):
    o_ref[...] = (
        jnp.dot(fc_ref[...], sw_ref[...], preferred_element_type=jnp.float32)
        * x_ref[...]
    )


def _tc1(filt_coeff, inputs, sw):
    blk = 4000
    grid = (NF // blk,)
    return pl.pallas_call(
        _tc1_body,
        grid=grid,
        in_specs=[
            pl.BlockSpec((blk, NB), lambda i: (i, 0)),
            pl.BlockSpec((blk, CIN), lambda i: (i, 0)),
            pl.BlockSpec((NB, CIN), lambda i: (0, 0)),
        ],
        out_specs=pl.BlockSpec((blk, CIN), lambda i: (i, 0)),
        out_shape=jax.ShapeDtypeStruct((NF, CIN), jnp.float32),
    )(filt_coeff, inputs, sw)


def _tc3a_body(agg_ref, nfc_ref, dw_ref, b_ref, s_ref):
    den = jnp.maximum(nfc_ref[...], 1).astype(jnp.float32)
    x = agg_ref[...] / den
    y = jnp.maximum(
        jnp.dot(x, dw_ref[...], preferred_element_type=jnp.float32) + b_ref[...],
        0.0,
    )
    ps = jnp.concatenate(
        [jnp.sum(y, axis=0)[None, :], jnp.sum(y * y, axis=0)[None, :],
         jnp.zeros((6, COUT), jnp.float32)], axis=0)

    @pl.when(pl.program_id(0) == 0)
    def _():
        s_ref[...] = ps

    @pl.when(pl.program_id(0) > 0)
    def _():
        s_ref[...] += ps


def _tc3a(agg, nf_count, dw, b):
    blk = 5000
    grid = (NV // blk,)
    return pl.pallas_call(
        _tc3a_body,
        grid=grid,
        in_specs=[
            pl.BlockSpec((blk, CIN), lambda i: (i, 0)),
            pl.BlockSpec((blk, 1), lambda i: (i, 0)),
            pl.BlockSpec((CIN, COUT), lambda i: (0, 0)),
            pl.BlockSpec((1, COUT), lambda i: (0, 0)),
        ],
        out_specs=pl.BlockSpec((8, COUT), lambda i: (0, 0)),
        out_shape=jax.ShapeDtypeStruct((8, COUT), jnp.float32),
    )(agg, nf_count.reshape(NV, 1), dw, b)


def _tc3b_body(agg_ref, nfc_ref, dw_ref, b_ref, s_ref, g_ref, bb_ref, o_ref):
    den = jnp.maximum(nfc_ref[...], 1).astype(jnp.float32)
    x = agg_ref[...] / den
    y = jnp.maximum(
        jnp.dot(x, dw_ref[...], preferred_element_type=jnp.float32) + b_ref[...],
        0.0,
    )
    inv_n = jnp.float32(1.0 / NV)
    mean = s_ref[0, :] * inv_n
    var = s_ref[1, :] * inv_n - mean * mean
    scale = g_ref[0, :] * lax.rsqrt(var + jnp.float32(1e-3))
    o_ref[...] = y * scale + (bb_ref[0, :] - mean * scale)


def _tc3b(agg, nf_count, dw, b, sums, gamma, beta):
    blk = 5000
    grid = (NV // blk,)
    return pl.pallas_call(
        _tc3b_body,
        grid=grid,
        in_specs=[
            pl.BlockSpec((blk, CIN), lambda i: (i, 0)),
            pl.BlockSpec((blk, 1), lambda i: (i, 0)),
            pl.BlockSpec((CIN, COUT), lambda i: (0, 0)),
            pl.BlockSpec((1, COUT), lambda i: (0, 0)),
            pl.BlockSpec((8, COUT), lambda i: (0, 0)),
            pl.BlockSpec((1, COUT), lambda i: (0, 0)),
            pl.BlockSpec((1, COUT), lambda i: (0, 0)),
        ],
        out_specs=pl.BlockSpec((blk, COUT), lambda i: (i, 0)),
        out_shape=jax.ShapeDtypeStruct((NV, COUT), jnp.float32),
    )(agg, nf_count.reshape(NV, 1), dw, b, sums, gamma, beta)


def kernel(inputs, face, nf_count, vt_map, filt_coeff, spatial_weights,
           depth_weights, biases, bn_gamma, bn_beta):
    del vt_map
    sw = spatial_weights.reshape(NB, CIN)
    filtered = _tc1(filt_coeff, inputs, sw)

    ev = jnp.concatenate(
        [face.T, jnp.full((3, COL_LEN - NF), jnp.int32(1 << 30), jnp.int32)],
        axis=1).reshape(-1)

    zeros = jnp.zeros((ACC_ROWS, CIN), jnp.float32)
    agg_pad = _sc_scatter(ev, filtered, zeros)

    sums = _tc3a(agg_pad, nf_count, depth_weights, biases)
    out = _tc3b(agg_pad, nf_count, depth_weights, biases, sums,
                bn_gamma.reshape(1, COUT), bn_beta.reshape(1, COUT))
    return out
